# baseline (speedup 1.0000x reference)
import functools

import jax
import jax.numpy as jnp
from jax import lax
from jax.experimental import pallas as pl
from jax.experimental.pallas import tpu as pltpu
from jax.experimental.pallas import tpu_sc as plsc

VOCAB = 100000
EMBED = 128
HIDDEN = 128
B = 4096
L = 200

NC = 2
NS = 16
NW = NC * NS
LANES = 16
NREG = EMBED // LANES
UNROLL = 8
NBUF = 3
HALF = B // 2

CHUNKS = ((0, 128), (128, 72))


def _make_sc_pool(nb):
    spw = nb // NW

    def body(x_hbm, table_hbm, out_hbm, xs_v, rows0_v, rows1_v, rows2_v,
             out_v, sem0, sem1, sem2):
        wid = lax.axis_index("s") * NC + lax.axis_index("c")
        base = wid * spw
        bufs = (rows0_v, rows1_v, rows2_v)
        sems = (sem0, sem1, sem2)

        pltpu.sync_copy(x_hbm.at[pl.ds(base, spw)], xs_v)

        def fire(s, slot):
            for off, size in CHUNKS:
                pltpu.async_copy(
                    table_hbm.at[xs_v.at[s, pl.ds(off, size)]],
                    bufs[slot].at[pl.ds(off, size)], sems[slot])

        def drain(slot):
            pltpu.make_async_copy(table_hbm.at[pl.ds(0, L)], bufs[slot],
                                  sems[slot]).wait()

        def reduce(slot, s):
            buf = bufs[slot]

            def red_body(i, accs):
                j0 = i * UNROLL
                for u in range(UNROLL):
                    accs = tuple(
                        accs[r] + buf[j0 + u, pl.ds(r * LANES, LANES)]
                        for r in range(NREG))
                return accs

            zero = jnp.zeros((LANES,), jnp.float32)
            accs = lax.fori_loop(0, L // UNROLL, red_body, (zero,) * NREG)
            for r in range(NREG):
                out_v[s, pl.ds(r * LANES, LANES)] = accs[r]

        fire(0, 0)
        fire(1, 1)

        def ring_body(g, _):
            s0 = NBUF * g
            for o in range(NBUF):
                s = s0 + o
                nxt = s + NBUF - 1

                @pl.when(nxt < spw)
                def _():
                    fire(nxt, (o + NBUF - 1) % NBUF)

                @pl.when(s < spw)
                def _():
                    drain(o)
                    reduce(o, s)
            return 0

        lax.fori_loop(0, (spw + NBUF - 1) // NBUF, ring_body, 0)

        pltpu.sync_copy(out_v, out_hbm.at[pl.ds(base, spw)])

    return functools.partial(
        pl.kernel,
        out_type=jax.ShapeDtypeStruct((nb, EMBED), jnp.float32),
        mesh=plsc.VectorSubcoreMesh(core_axis_name="c", subcore_axis_name="s"),
        compiler_params=pltpu.CompilerParams(needs_layout_passes=False),
        scratch_types=[
            pltpu.VMEM((spw, L), jnp.int32),
            pltpu.VMEM((L, EMBED), jnp.float32),
            pltpu.VMEM((L, EMBED), jnp.float32),
            pltpu.VMEM((L, EMBED), jnp.float32),
            pltpu.VMEM((spw, EMBED), jnp.float32),
            pltpu.SemaphoreType.DMA,
            pltpu.SemaphoreType.DMA,
            pltpu.SemaphoreType.DMA,
        ],
    )(body)


_sc_pool_half = _make_sc_pool(HALF)


def _mlp_body(p_ref, w1_ref, b1_ref, w2_ref, b2_ref, o_ref):
    p = p_ref[...] * jnp.float32(1.0 / L)
    h = jnp.dot(p, w1_ref[...], preferred_element_type=jnp.float32)
    h = jnp.maximum(h + b1_ref[...], 0.0)
    o_ref[...] = jnp.sum(h * w2_ref[...], axis=1, keepdims=True) + b2_ref[...]


def _mlp(pooled, W1, b1, W2, b2):
    return pl.pallas_call(
        _mlp_body,
        out_shape=jax.ShapeDtypeStruct((pooled.shape[0], 1), jnp.float32),
    )(pooled, W1, b1.reshape(1, HIDDEN), W2.reshape(1, HIDDEN),
      b2.reshape(1, 1))


def kernel(x, table, W1, b1, W2, b2):
    p1 = _sc_pool_half(x[:HALF], table)
    p2 = _sc_pool_half(x[HALF:], table)
    o1 = _mlp(p1, W1, b1, W2, b2)
    o2 = _mlp(p2, W1, b1, W2, b2)
    return jnp.concatenate([o1, o2], axis=0).reshape(B)


# speedup vs baseline: 1.0055x; 1.0055x over previous
import functools

import jax
import jax.numpy as jnp
from jax import lax
from jax.experimental import pallas as pl
from jax.experimental.pallas import tpu as pltpu
from jax.experimental.pallas import tpu_sc as plsc

VOCAB = 100000
EMBED = 128
HIDDEN = 128
B = 4096
L = 200

NC = 2
NS = 16
NW = NC * NS
SPW = B // NW
LANES = 16
NREG = EMBED // LANES
UNROLL = 8
NBUF = 4

CHUNKS = ((0, 128), (128, 72))


def _sc_pool_body(x_hbm, table_hbm, out_hbm, xs_v,
                  rows0_v, rows1_v, rows2_v, rows3_v,
                  orow0_v, orow1_v, orow2_v, orow3_v,
                  sem0, sem1, sem2, sem3,
                  osem0, osem1, osem2, osem3):
    wid = lax.axis_index("s") * NC + lax.axis_index("c")
    base = wid * SPW
    bufs = (rows0_v, rows1_v, rows2_v, rows3_v)
    sems = (sem0, sem1, sem2, sem3)
    orows = (orow0_v, orow1_v, orow2_v, orow3_v)
    osems = (osem0, osem1, osem2, osem3)

    pltpu.sync_copy(x_hbm.at[pl.ds(base * L, SPW * L)], xs_v)

    def fire(s, slot):
        for off, size in CHUNKS:
            pltpu.async_copy(
                table_hbm.at[xs_v.at[pl.ds(s * L + off, size)]],
                bufs[slot].at[pl.ds(off, size)], sems[slot])

    def drain(slot):
        pltpu.make_async_copy(table_hbm.at[pl.ds(0, L)], bufs[slot],
                              sems[slot]).wait()

    def reduce_store(slot, s):
        buf = bufs[slot]

        def red_body(i, accs):
            j0 = i * UNROLL
            for u in range(UNROLL):
                accs = tuple(accs[r] + buf[j0 + u, pl.ds(r * LANES, LANES)]
                             for r in range(NREG))
            return accs

        zero = jnp.zeros((LANES,), jnp.float32)
        accs = lax.fori_loop(0, L // UNROLL, red_body, (zero,) * NREG)

        @pl.when(s >= NBUF)
        def _():
            pltpu.make_async_copy(out_hbm.at[base], orows[slot],
                                  osems[slot]).wait()
        for r in range(NREG):
            orows[slot][pl.ds(r * LANES, LANES)] = accs[r]
        pltpu.async_copy(orows[slot], out_hbm.at[base + s], osems[slot])

    fire(0, 0)
    fire(1, 1)
    fire(2, 2)

    def ring_body(g, _):
        s0 = NBUF * g
        for o in range(NBUF):
            s = s0 + o
            nxt = s + NBUF - 1

            @pl.when(nxt < SPW)
            def _():
                fire(nxt, (o + NBUF - 1) % NBUF)

            drain(o)
            reduce_store(o, s)
        return 0

    lax.fori_loop(0, SPW // NBUF, ring_body, 0)

    for slot in range(NBUF):
        pltpu.make_async_copy(out_hbm.at[base], orows[slot],
                              osems[slot]).wait()


_sc_pool = functools.partial(
    pl.kernel,
    out_type=jax.ShapeDtypeStruct((B, EMBED), jnp.float32),
    mesh=plsc.VectorSubcoreMesh(core_axis_name="c", subcore_axis_name="s"),
    compiler_params=pltpu.CompilerParams(needs_layout_passes=False),
    scratch_types=[
        pltpu.VMEM((SPW * L,), jnp.int32),
        pltpu.VMEM((L, EMBED), jnp.float32),
        pltpu.VMEM((L, EMBED), jnp.float32),
        pltpu.VMEM((L, EMBED), jnp.float32),
        pltpu.VMEM((L, EMBED), jnp.float32),
        pltpu.VMEM((EMBED,), jnp.float32),
        pltpu.VMEM((EMBED,), jnp.float32),
        pltpu.VMEM((EMBED,), jnp.float32),
        pltpu.VMEM((EMBED,), jnp.float32),
        pltpu.SemaphoreType.DMA,
        pltpu.SemaphoreType.DMA,
        pltpu.SemaphoreType.DMA,
        pltpu.SemaphoreType.DMA,
        pltpu.SemaphoreType.DMA,
        pltpu.SemaphoreType.DMA,
        pltpu.SemaphoreType.DMA,
        pltpu.SemaphoreType.DMA,
    ],
)(_sc_pool_body)


def _mlp_body(p_ref, w1_ref, b1_ref, w2_ref, b2_ref, o_ref):
    p = p_ref[...] * jnp.float32(1.0 / L)
    h = jnp.dot(p, w1_ref[...], preferred_element_type=jnp.float32)
    h = jnp.maximum(h + b1_ref[...], 0.0)
    o_ref[...] = jnp.sum(h * w2_ref[...], axis=1, keepdims=True) + b2_ref[...]


def kernel(x, table, W1, b1, W2, b2):
    pooled_sum = _sc_pool(x.reshape(B * L), table)
    out = pl.pallas_call(
        _mlp_body,
        out_shape=jax.ShapeDtypeStruct((B, 1), jnp.float32),
    )(pooled_sum, W1, b1.reshape(1, HIDDEN), W2.reshape(1, HIDDEN),
      b2.reshape(1, 1))
    return out.reshape(B)


# speedup vs baseline: 1.0439x; 1.0382x over previous
import functools

import jax
import jax.numpy as jnp
from jax import lax
from jax.experimental import pallas as pl
from jax.experimental.pallas import tpu as pltpu
from jax.experimental.pallas import tpu_sc as plsc

VOCAB = 100000
EMBED = 128
HIDDEN = 128
B = 4096
L = 200

NC = 2
NS = 16
NW = NC * NS
SPW = B // NW
LANES = 16
NREG = EMBED // LANES
UNROLL = 8
NBUF = 3

CHUNKS = ((0, 128), (128, 72))


def _sc_pool_body(x_hbm, table_hbm, out_hbm, xs_v, rows0_v, rows1_v, rows2_v,
                  out_v, sem0, sem1, sem2):
    wid = lax.axis_index("s") * NC + lax.axis_index("c")
    base = wid * SPW
    bufs = (rows0_v, rows1_v, rows2_v)
    sems = (sem0, sem1, sem2)

    pltpu.sync_copy(x_hbm.at[pl.ds(base, SPW)], xs_v)

    def fire(s, slot):
        for off, size in CHUNKS:
            pltpu.async_copy(
                table_hbm.at[xs_v.at[s, pl.ds(off, size)]],
                bufs[slot].at[pl.ds(off, size)], sems[slot])

    def drain(slot):
        pltpu.make_async_copy(table_hbm.at[pl.ds(0, L)], bufs[slot],
                              sems[slot]).wait()

    def reduce(slot, s):
        buf = bufs[slot]

        def red_body(i, accs):
            j0 = i * UNROLL
            for u in range(UNROLL):
                accs = tuple(accs[r] + buf[j0 + u, pl.ds(r * LANES, LANES)]
                             for r in range(NREG))
            return accs

        zero = jnp.zeros((LANES,), jnp.float32)
        accs = lax.fori_loop(0, L // UNROLL, red_body, (zero,) * NREG)
        for r in range(NREG):
            out_v[s, pl.ds(r * LANES, LANES)] = accs[r]

    fire(0, 0)
    fire(1, 1)

    def ring_body(g, _):
        s0 = NBUF * g
        for o in range(NBUF):
            s = s0 + o
            nxt = s + NBUF - 1

            @pl.when(nxt < SPW)
            def _():
                fire(nxt, (o + NBUF - 1) % NBUF)

            @pl.when(s < SPW)
            def _():
                drain(o)
                reduce(o, s)
        return 0

    lax.fori_loop(0, (SPW + NBUF - 1) // NBUF, ring_body, 0)

    pltpu.sync_copy(out_v, out_hbm.at[pl.ds(base, SPW)])


_sc_pool = functools.partial(
    pl.kernel,
    out_type=jax.ShapeDtypeStruct((B, EMBED), jnp.float32),
    mesh=plsc.VectorSubcoreMesh(core_axis_name="c", subcore_axis_name="s"),
    compiler_params=pltpu.CompilerParams(needs_layout_passes=False),
    scratch_types=[
        pltpu.VMEM((SPW, L), jnp.int32),
        pltpu.VMEM((L, EMBED), jnp.float32),
        pltpu.VMEM((L, EMBED), jnp.float32),
        pltpu.VMEM((L, EMBED), jnp.float32),
        pltpu.VMEM((SPW, EMBED), jnp.float32),
        pltpu.SemaphoreType.DMA,
        pltpu.SemaphoreType.DMA,
        pltpu.SemaphoreType.DMA,
    ],
)(_sc_pool_body)


def _mlp_body(p_ref, w1_ref, b1_ref, w2_ref, b2_ref, o_ref):
    p = p_ref[...] * jnp.float32(1.0 / L)
    h = jnp.dot(p, w1_ref[...], preferred_element_type=jnp.float32)
    h = jnp.maximum(h + b1_ref[...], 0.0)
    o_ref[...] = jnp.sum(h * w2_ref[...], axis=1, keepdims=True) + b2_ref[...]


def kernel(x, table, W1, b1, W2, b2):
    pooled_sum = _sc_pool(x, table)
    out = pl.pallas_call(
        _mlp_body,
        out_shape=jax.ShapeDtypeStruct((B, 1), jnp.float32),
    )(pooled_sum, W1, b1.reshape(1, HIDDEN), W2.reshape(1, HIDDEN),
      b2.reshape(1, 1))
    return out.reshape(B)
